# baseline (speedup 1.0000x reference)
import functools

import jax
import jax.numpy as jnp
from jax import lax
from jax.experimental import pallas as pl
from jax.experimental.pallas import tpu as pltpu
from jax.experimental.pallas import tpu_sc as plsc

_NUM_CLASSES = 1000000
_FEAT_DIM = 64
_BATCH = 16384
_LAMBDA_C = 0.01
_EPS = 1e-12

_NC = 2
_NS = 16
_NW = _NC * _NS
_B_PER_W = _BATCH // _NW
_IDX_CHUNK = 128
_N_CHUNKS = _B_PER_W // _IDX_CHUNK


def _gather_body(labels_hbm, table_hbm, out_hbm, idx_v, rows_v, sem):
    wid = lax.axis_index("s") * _NC + lax.axis_index("c")
    base = wid * _B_PER_W
    pltpu.sync_copy(labels_hbm.at[pl.ds(base, _B_PER_W)], idx_v)
    copies = []
    for j in range(_N_CHUNKS):
        sl = pl.ds(j * _IDX_CHUNK, _IDX_CHUNK)
        copies.append(
            pltpu.async_copy(table_hbm.at[idx_v.at[sl]], rows_v.at[sl], sem)
        )
    for c in copies:
        c.wait()
    pltpu.sync_copy(rows_v, out_hbm.at[pl.ds(base, _B_PER_W)])


_gather = pl.kernel(
    _gather_body,
    out_type=jax.ShapeDtypeStruct((_BATCH, _FEAT_DIM), jnp.float32),
    mesh=plsc.VectorSubcoreMesh(core_axis_name="c", subcore_axis_name="s"),
    scratch_types=[
        pltpu.VMEM((_B_PER_W,), jnp.int32),
        pltpu.VMEM((_B_PER_W, _FEAT_DIM), jnp.float32),
        pltpu.SemaphoreType.DMA,
    ],
    compiler_params=pltpu.CompilerParams(use_tc_tiling_on_sc=False),
)


_ROW_BLOCK = 2048
_N_BLOCKS = _BATCH // _ROW_BLOCK


def _loss_body(f_ref, c_ref, o_ref):
    f = f_ref[...]
    c = c_ref[...]
    ff = jnp.sum(f * f, axis=1)
    cc = jnp.sum(c * c, axis=1)
    fc = jnp.sum(f * c, axis=1)
    e2 = jnp.float32(_EPS * _EPS)
    mf2 = jnp.maximum(ff, e2)
    mc2 = jnp.maximum(cc, e2)
    dist = ff / mf2 + cc / mc2 - 2.0 * fc * lax.rsqrt(mf2 * mc2)
    part = (jnp.sum(dist) * jnp.float32(_LAMBDA_C / _BATCH)).reshape(1, 1)

    @pl.when(pl.program_id(0) == 0)
    def _():
        o_ref[...] = jnp.zeros_like(o_ref)

    o_ref[...] += part


_loss = pl.pallas_call(
    _loss_body,
    grid=(_N_BLOCKS,),
    in_specs=[
        pl.BlockSpec((_ROW_BLOCK, _FEAT_DIM), lambda i: (i, 0)),
        pl.BlockSpec((_ROW_BLOCK, _FEAT_DIM), lambda i: (i, 0)),
    ],
    out_specs=pl.BlockSpec((1, 1), lambda i: (0, 0)),
    out_shape=jax.ShapeDtypeStruct((1, 1), jnp.float32),
)


def kernel(features, labels, centers):
    cb = _gather(labels.astype(jnp.int32), centers)
    return _loss(features, cb)[0, 0]


# speedup vs baseline: 1.5756x; 1.5756x over previous
import functools

import jax
import jax.numpy as jnp
from jax import lax
from jax.experimental import pallas as pl
from jax.experimental.pallas import tpu as pltpu
from jax.experimental.pallas import tpu_sc as plsc

_NUM_CLASSES = 1000000
_FEAT_DIM = 64
_BATCH = 16384
_LAMBDA_C = 0.01
_EPS = 1e-12

_NC = 2
_NS = 16
_NW = _NC * _NS
_B_PER_W = _BATCH // _NW
_CHUNK = 128
_N_CHUNKS = _B_PER_W // _CHUNK
_LANES = 16


def _rsqrt16(x):
    i = plsc.bitcast(x, jnp.int32)
    y = plsc.bitcast(jnp.int32(0x5F3759DF) - (i >> 1), jnp.float32)
    for _ in range(3):
        y = y * (1.5 - 0.5 * x * y * y)
    return y


def _sc_body(features_hbm, labels_hbm, table_hbm, out_hbm,
             lbl_v, rows_v, feat_v, acc_v, sem):
    wid = lax.axis_index("s") * _NC + lax.axis_index("c")
    base = wid * _B_PER_W
    pltpu.sync_copy(labels_hbm.at[pl.ds(base, _B_PER_W)], lbl_v)

    def fetch_group(g, carry):
        base16 = g * _LANES
        lbl16 = lbl_v[pl.ds(base16, _LANES)]
        for j in range(_LANES):
            pltpu.async_copy(table_hbm.at[pl.ds(lbl16[j], 1)],
                             rows_v.at[pl.ds(base16 + j, 1)], sem)
        return carry

    lax.fori_loop(0, _B_PER_W // _LANES, fetch_group, 0)

    acc_v[...] = jnp.zeros((_LANES,), jnp.float32)
    iota = lax.broadcasted_iota(jnp.int32, (_LANES,), 0)
    e2 = jnp.float32(_EPS * _EPS)

    pltpu.make_async_copy(table_hbm.at[pl.ds(0, _B_PER_W)], rows_v, sem).wait()

    for ch in range(_N_CHUNKS):
        row0 = ch * _CHUNK
        pltpu.sync_copy(features_hbm.at[pl.ds(base + row0, _CHUNK)], feat_v)
        for g in range(_CHUNK // _LANES):
            loc16 = iota + (g * _LANES)
            glob16 = loc16 + row0

            def dim_step(d, carry):
                ff, cc, fc = carry
                dvec = jnp.full((_LANES,), d, jnp.int32)
                f = plsc.load_gather(feat_v, [loc16, dvec])
                c = plsc.load_gather(rows_v, [glob16, dvec])
                return (ff + f * f, cc + c * c, fc + f * c)

            zero = jnp.zeros((_LANES,), jnp.float32)
            ff, cc, fc = lax.fori_loop(0, _FEAT_DIM, dim_step,
                                       (zero, zero, zero))
            mf2 = jnp.maximum(ff, e2)
            mc2 = jnp.maximum(cc, e2)
            dist = ff / mf2 + cc / mc2 - 2.0 * fc * _rsqrt16(mf2 * mc2)
            acc_v[...] += dist
    pltpu.sync_copy(acc_v, out_hbm.at[pl.ds(wid * _LANES, _LANES)])


_sc_loss = pl.kernel(
    _sc_body,
    out_type=jax.ShapeDtypeStruct((_NW * _LANES,), jnp.float32),
    mesh=plsc.VectorSubcoreMesh(core_axis_name="c", subcore_axis_name="s"),
    scratch_types=[
        pltpu.VMEM((_B_PER_W,), jnp.int32),
        pltpu.VMEM((_B_PER_W, _FEAT_DIM), jnp.float32),
        pltpu.VMEM((_CHUNK, _FEAT_DIM), jnp.float32),
        pltpu.VMEM((_LANES,), jnp.float32),
        pltpu.SemaphoreType.DMA,
    ],
    compiler_params=pltpu.CompilerParams(needs_layout_passes=False),
)


def _finish_body(p_ref, o_ref):
    o_ref[...] = (jnp.sum(p_ref[...]) * jnp.float32(_LAMBDA_C / _BATCH)
                  ).reshape(1, 1)


_finish = pl.pallas_call(
    _finish_body,
    out_shape=jax.ShapeDtypeStruct((1, 1), jnp.float32),
)


def kernel(features, labels, centers):
    partials = _sc_loss(features, labels.astype(jnp.int32), centers)
    return _finish(partials.reshape(_NW, _LANES))[0, 0]


# speedup vs baseline: 1.7030x; 1.0808x over previous
import functools

import jax
import jax.numpy as jnp
from jax import lax
from jax.experimental import pallas as pl
from jax.experimental.pallas import tpu as pltpu
from jax.experimental.pallas import tpu_sc as plsc

_NUM_CLASSES = 1000000
_FEAT_DIM = 64
_BATCH = 16384
_LAMBDA_C = 0.01
_EPS = 1e-12

_NC = 2
_NS = 16
_NW = _NC * _NS
_B_PER_W = _BATCH // _NW
_LANES = 16


def _gather_body(labels_hbm, table_hbm, out_hbm, lbl_v, rows_v, sem):
    wid = lax.axis_index("s") * _NC + lax.axis_index("c")
    base = wid * _B_PER_W
    pltpu.sync_copy(labels_hbm.at[pl.ds(base, _B_PER_W)], lbl_v)

    def fetch_group(g, carry):
        base16 = g * _LANES
        lbl16 = lbl_v[pl.ds(base16, _LANES)]
        for j in range(_LANES):
            pltpu.async_copy(table_hbm.at[pl.ds(lbl16[j], 1)],
                             rows_v.at[pl.ds(base16 + j, 1)], sem)
        return carry

    lax.fori_loop(0, _B_PER_W // _LANES, fetch_group, 0)
    pltpu.make_async_copy(table_hbm.at[pl.ds(0, _B_PER_W)], rows_v, sem).wait()
    pltpu.sync_copy(rows_v, out_hbm.at[pl.ds(base, _B_PER_W)])


_sc_gather = pl.kernel(
    _gather_body,
    out_type=jax.ShapeDtypeStruct((_BATCH, _FEAT_DIM), jnp.float32),
    mesh=plsc.VectorSubcoreMesh(core_axis_name="c", subcore_axis_name="s"),
    scratch_types=[
        pltpu.VMEM((_B_PER_W,), jnp.int32),
        pltpu.VMEM((_B_PER_W, _FEAT_DIM), jnp.float32),
        pltpu.SemaphoreType.DMA,
    ],
)


_ROW_BLOCK = 2048
_N_BLOCKS = _BATCH // _ROW_BLOCK


def _loss_body(f_ref, c_ref, o_ref):
    f = f_ref[...]
    c = c_ref[...]
    ff = jnp.sum(f * f, axis=1)
    cc = jnp.sum(c * c, axis=1)
    fc = jnp.sum(f * c, axis=1)
    e2 = jnp.float32(_EPS * _EPS)
    mf2 = jnp.maximum(ff, e2)
    mc2 = jnp.maximum(cc, e2)
    dist = ff / mf2 + cc / mc2 - 2.0 * fc * lax.rsqrt(mf2 * mc2)
    part = (jnp.sum(dist) * jnp.float32(_LAMBDA_C / _BATCH)).reshape(1, 1)

    @pl.when(pl.program_id(0) == 0)
    def _():
        o_ref[...] = jnp.zeros_like(o_ref)

    o_ref[...] += part


_loss = pl.pallas_call(
    _loss_body,
    grid=(_N_BLOCKS,),
    in_specs=[
        pl.BlockSpec((_ROW_BLOCK, _FEAT_DIM), lambda i: (i, 0)),
        pl.BlockSpec((_ROW_BLOCK, _FEAT_DIM), lambda i: (i, 0)),
    ],
    out_specs=pl.BlockSpec((1, 1), lambda i: (0, 0)),
    out_shape=jax.ShapeDtypeStruct((1, 1), jnp.float32),
)


def kernel(features, labels, centers):
    cb = _sc_gather(labels.astype(jnp.int32), centers)
    return _loss(features, cb)[0, 0]
